# Initial kernel scaffold; baseline (speedup 1.0000x reference)
import jax
import jax.numpy as jnp
from jax.experimental import pallas as pl


def kernel(teacher_logits_a, teacher_logits_b, student_results, span_a_selected_index, span_b_selected_index, span_c_a_selected_index, span_c_b_selected_index):
    raise NotImplementedError("write your pallas kernel here")


# speedup vs baseline: 1.8079x; 1.8079x over previous
import functools

import jax
import jax.numpy as jnp
from jax import lax
from jax.experimental import pallas as pl
from jax.experimental.pallas import tpu as pltpu
from jax.experimental.pallas import tpu_sc as plsc

B, L, D, K = 4, 8192, 768, 1024
NW = 32
RPW = (B * K) // NW
LANES = 16


def _body(ta, tb, st, ia, ib, ica, icb, out_s, out_t, idx_v, rows_v, sem):
    wid = lax.axis_index("s") * 2 + lax.axis_index("c")
    flat_base = pl.multiple_of(wid * RPW, RPW)
    b = flat_base // K
    boff = b * L
    k_base = flat_base - b * K
    out_bb = b * (2 * K)

    jobs = (
        (ta, ia, out_t, 0),
        (tb, ib, out_t, K),
        (st, ica, out_s, 0),
        (st, icb, out_s, K),
    )
    for tab, iref, oref, joff in jobs:
        pltpu.sync_copy(iref.at[pl.ds(flat_base, RPW)], idx_v)
        for i in range(RPW // LANES):
            sl = pl.ds(i * LANES, LANES)
            idx_v[sl] = idx_v[sl] + boff
        pltpu.async_copy(tab.at[idx_v], rows_v, sem).wait()
        out_base = pl.multiple_of(out_bb + joff + k_base, RPW)
        pltpu.sync_copy(rows_v, oref.at[pl.ds(out_base, RPW)])


@functools.partial(
    pl.kernel,
    mesh=plsc.VectorSubcoreMesh(core_axis_name="c", subcore_axis_name="s"),
    out_type=[
        jax.ShapeDtypeStruct((B * 2 * K, D), jnp.float32),
        jax.ShapeDtypeStruct((B * 2 * K, D), jnp.float32),
    ],
    scratch_types=[
        pltpu.VMEM((RPW,), jnp.int32),
        pltpu.VMEM((RPW, D), jnp.float32),
        pltpu.SemaphoreType.DMA,
    ],
)
def _gather(ta, tb, st, ia, ib, ica, icb, out_s, out_t, idx_v, rows_v, sem):
    _body(ta, tb, st, ia, ib, ica, icb, out_s, out_t, idx_v, rows_v, sem)


def kernel(teacher_logits_a, teacher_logits_b, student_results,
           span_a_selected_index, span_b_selected_index,
           span_c_a_selected_index, span_c_b_selected_index):
    ta = teacher_logits_a.reshape(B * L, D)
    tb = teacher_logits_b.reshape(B * L, D)
    st = student_results.reshape(B * L, D)
    ia = span_a_selected_index.reshape(B * K).astype(jnp.int32)
    ib = span_b_selected_index.reshape(B * K).astype(jnp.int32)
    ica = span_c_a_selected_index.reshape(B * K).astype(jnp.int32)
    icb = span_c_b_selected_index.reshape(B * K).astype(jnp.int32)
    out_s, out_t = _gather(ta, tb, st, ia, ib, ica, icb)
    return (out_s.reshape(B, 2 * K, D), out_t.reshape(B, 2 * K, D))
